# Initial kernel scaffold; baseline (speedup 1.0000x reference)
import jax
import jax.numpy as jnp
from jax.experimental import pallas as pl


def kernel(x, ei, batch, W1a, b1a, W1b, b1b, g1, be1, W2a, b2a, W2b, b2b, g2, be2):
    raise NotImplementedError("write your pallas kernel here")


# speedup vs baseline: 5.3811x; 5.3811x over previous
import functools

import jax
import jax.numpy as jnp
import numpy as np
from jax import lax
from jax.experimental import pallas as pl
from jax.experimental.pallas import tpu as pltpu
from jax.experimental.pallas import tpu_sc as plsc

N, E, F_IN, H, G = 10000, 320000, 128, 64, 64
BN_EPS = 1e-5

NC, NS = 2, 16
NW = NC * NS
CHUNK = 128
ROWS_PER_W = 80
ROWS = NW * ROWS_PER_W
E_PAD = ROWS * CHUNK
N_ACC = 10112
ZROWS = N_ACC // NS


def _seg_body(y_hbm, src_hbm, dst_hbm, out_hbm, src_v, dst_v, rows_v, zbuf, acc, sem):
    c = lax.axis_index("c")
    s = lax.axis_index("s")
    wid = c * NS + s

    def _zrow(r, carry):
        for k in range(H // 16):
            zbuf[r, pl.ds(16 * k, 16)] = jnp.zeros((16,), jnp.float32)
        return carry

    lax.fori_loop(0, ZROWS, _zrow, 0)
    pltpu.sync_copy(zbuf, acc.at[pl.ds(s * ZROWS, ZROWS)])

    base = wid * ROWS_PER_W
    pltpu.sync_copy(src_hbm.at[pl.ds(base, ROWS_PER_W)], src_v)
    pltpu.sync_copy(dst_hbm.at[pl.ds(base, ROWS_PER_W)], dst_v)
    plsc.subcore_barrier()

    def _step(j, carry):
        pltpu.async_copy(y_hbm.at[src_v.at[j]], rows_v, sem).wait()
        pltpu.sync_copy(rows_v, acc.at[dst_v.at[j]], add=True)
        return carry

    lax.fori_loop(0, ROWS_PER_W, _step, 0)
    plsc.subcore_barrier()

    pltpu.sync_copy(acc.at[pl.ds(s * ZROWS, ZROWS)], zbuf)
    pltpu.sync_copy(zbuf, out_hbm.at[c].at[pl.ds(s * ZROWS, ZROWS)])


_seg_sum = pl.kernel(
    _seg_body,
    out_type=jax.ShapeDtypeStruct((NC, N_ACC, H), jnp.float32),
    mesh=plsc.VectorSubcoreMesh(
        core_axis_name="c", subcore_axis_name="s", num_cores=NC, num_subcores=NS
    ),
    scratch_types=[
        pltpu.VMEM((ROWS_PER_W, CHUNK), jnp.int32),
        pltpu.VMEM((ROWS_PER_W, CHUNK), jnp.int32),
        pltpu.VMEM((CHUNK, H), jnp.float32),
        pltpu.VMEM((ZROWS, H), jnp.float32),
        pltpu.VMEM_SHARED((N_ACC, H), jnp.float32),
        pltpu.SemaphoreType.DMA,
    ],
    compiler_params=pltpu.CompilerParams(use_tc_tiling_on_sc=False),
)

BLK = 1000
GRID = N // BLK


def _proj_body(x_ref, w_ref, o_ref):
    o_ref[:] = jnp.dot(x_ref[:], w_ref[:], preferred_element_type=jnp.float32)


_proj = pl.pallas_call(
    _proj_body,
    grid=(GRID,),
    in_specs=[
        pl.BlockSpec((BLK, F_IN), lambda i: (i, 0)),
        pl.BlockSpec((F_IN, H), lambda i: (0, 0)),
    ],
    out_specs=pl.BlockSpec((BLK, H), lambda i: (i, 0)),
    out_shape=jax.ShapeDtypeStruct((N, H), jnp.float32),
)


def _mlp1_body(y_ref, a0_ref, a1_ref, ba_ref, w_ref, bb_ref, g_ref, be_ref, o_ref):
    t = jnp.maximum(y_ref[:] + a0_ref[:] + a1_ref[:] + ba_ref[:], 0.0)
    h = jnp.dot(t, w_ref[:], preferred_element_type=jnp.float32) + bb_ref[:]
    scale = g_ref[:] * (1.0 / np.sqrt(1.0 + BN_EPS))
    o_ref[:] = jnp.maximum(h * scale + be_ref[:], 0.0)


_row = lambda i: (i, 0)
_fix = lambda i: (0, 0)

_mlp1 = pl.pallas_call(
    _mlp1_body,
    grid=(GRID,),
    in_specs=[
        pl.BlockSpec((BLK, H), _row),
        pl.BlockSpec((BLK, H), _row),
        pl.BlockSpec((BLK, H), _row),
        pl.BlockSpec((1, H), _fix),
        pl.BlockSpec((H, H), _fix),
        pl.BlockSpec((1, H), _fix),
        pl.BlockSpec((1, H), _fix),
        pl.BlockSpec((1, H), _fix),
    ],
    out_specs=pl.BlockSpec((BLK, H), _row),
    out_shape=jax.ShapeDtypeStruct((N, H), jnp.float32),
)


def _mlp2_body(h_ref, a0_ref, a1_ref, bid_ref, wa_ref, ba_ref, wb_ref, bb_ref,
               g_ref, be_ref, o_ref, acc_s, acc_c):
    j = pl.program_id(0)

    @pl.when(j == 0)
    def _():
        acc_s[:] = jnp.zeros_like(acc_s)
        acc_c[:] = jnp.zeros_like(acc_c)

    u = h_ref[:] + a0_ref[:] + a1_ref[:]
    t = jnp.maximum(jnp.dot(u, wa_ref[:], preferred_element_type=jnp.float32) + ba_ref[:], 0.0)
    h2 = jnp.dot(t, wb_ref[:], preferred_element_type=jnp.float32) + bb_ref[:]
    scale = g_ref[:] * (1.0 / np.sqrt(1.0 + BN_EPS))
    h2 = jnp.maximum(h2 * scale + be_ref[:], 0.0)

    oh = (bid_ref[:] == lax.broadcasted_iota(jnp.int32, (1, G), 1)).astype(jnp.float32)
    dims = (((0,), (0,)), ((), ()))
    acc_s[:] += lax.dot_general(oh, h2, dims, preferred_element_type=jnp.float32)
    acc_c[:] += lax.dot_general(oh, jnp.ones_like(h2), dims, preferred_element_type=jnp.float32)

    @pl.when(j == pl.num_programs(0) - 1)
    def _():
        o_ref[:] = acc_s[:] / jnp.maximum(acc_c[:], 1.0)


_mlp2_pool = pl.pallas_call(
    _mlp2_body,
    grid=(GRID,),
    in_specs=[
        pl.BlockSpec((BLK, H), _row),
        pl.BlockSpec((BLK, H), _row),
        pl.BlockSpec((BLK, H), _row),
        pl.BlockSpec((BLK, 1), _row),
        pl.BlockSpec((H, H), _fix),
        pl.BlockSpec((1, H), _fix),
        pl.BlockSpec((H, H), _fix),
        pl.BlockSpec((1, H), _fix),
        pl.BlockSpec((1, H), _fix),
        pl.BlockSpec((1, H), _fix),
    ],
    out_specs=pl.BlockSpec((G, H), _fix),
    out_shape=jax.ShapeDtypeStruct((G, H), jnp.float32),
    scratch_shapes=[
        pltpu.VMEM((G, H), jnp.float32),
        pltpu.VMEM((G, H), jnp.float32),
    ],
)


def kernel(x, ei, batch, W1a, b1a, W1b, b1b, g1, be1, W2a, b2a, W2b, b2b, g2, be2):
    src, dst = ei[0], ei[1]
    pad = E_PAD - E
    src_p = jnp.concatenate([src, jnp.zeros((pad,), jnp.int32)]).reshape(ROWS, CHUNK)
    dst_p = jnp.concatenate([dst, jnp.full((pad,), N, jnp.int32)]).reshape(ROWS, CHUNK)

    y = _proj(x, W1a)
    a1 = _seg_sum(y, src_p, dst_p)
    h1 = _mlp1(y, a1[0, :N], a1[1, :N], b1a.reshape(1, H), W1b, b1b.reshape(1, H),
               g1.reshape(1, H), be1.reshape(1, H))
    a2 = _seg_sum(h1, src_p, dst_p)
    out = _mlp2_pool(h1, a2[0, :N], a2[1, :N], batch.reshape(N, 1), W2a,
                     b2a.reshape(1, H), W2b, b2b.reshape(1, H),
                     g2.reshape(1, H), be2.reshape(1, H))
    return out


# speedup vs baseline: 5.8369x; 1.0847x over previous
import functools

import jax
import jax.numpy as jnp
import numpy as np
from jax import lax
from jax.experimental import pallas as pl
from jax.experimental.pallas import tpu as pltpu
from jax.experimental.pallas import tpu_sc as plsc

N, E, F_IN, H, G = 10000, 320000, 128, 64, 64
BN_EPS = 1e-5

NC, NS = 2, 16
NW = NC * NS
CHUNK = 128
ROWS_PER_W = 80
ROWS = NW * ROWS_PER_W
E_PAD = ROWS * CHUNK
N_ACC = 10112
ZROWS = N_ACC // NS


def _seg_body(y_hbm, src_hbm, dst_hbm, out_hbm, src_v, dst_v, rows_v, zbuf, acc, sem0, sem1):
    c = lax.axis_index("c")
    s = lax.axis_index("s")
    wid = c * NS + s

    def _zrow(r, carry):
        for k in range(H // 16):
            zbuf[r, pl.ds(16 * k, 16)] = jnp.zeros((16,), jnp.float32)
        return carry

    lax.fori_loop(0, ZROWS, _zrow, 0)
    pltpu.sync_copy(zbuf, acc.at[pl.ds(s * ZROWS, ZROWS)])

    base = wid * ROWS_PER_W
    pltpu.sync_copy(src_hbm.at[pl.ds(base, ROWS_PER_W)], src_v)
    pltpu.sync_copy(dst_hbm.at[pl.ds(base, ROWS_PER_W)], dst_v)
    plsc.subcore_barrier()

    sems = (sem0, sem1)
    bufs = (rows_v.at[0], rows_v.at[1])

    def _gather(j, b):
        pltpu.make_async_copy(y_hbm.at[src_v.at[j]], bufs[b], sems[b]).start()

    def _gwait(b):
        pltpu.make_async_copy(y_hbm.at[src_v.at[0]], bufs[b], sems[b]).wait()

    _gather(0, 0)

    def _step(j2, carry):
        for b in range(2):
            j = 2 * j2 + b
            _gwait(b)
            nxt = j + 1

            @pl.when(nxt < ROWS_PER_W)
            def _():
                _gather(nxt, 1 - b)

            pltpu.sync_copy(bufs[b], acc.at[dst_v.at[j]], add=True)
        return carry

    lax.fori_loop(0, ROWS_PER_W // 2, _step, 0)
    plsc.subcore_barrier()

    pltpu.sync_copy(acc.at[pl.ds(s * ZROWS, ZROWS)], out_hbm.at[c].at[pl.ds(s * ZROWS, ZROWS)])


_seg_sum = pl.kernel(
    _seg_body,
    out_type=jax.ShapeDtypeStruct((NC, N_ACC, H), jnp.float32),
    mesh=plsc.VectorSubcoreMesh(
        core_axis_name="c", subcore_axis_name="s", num_cores=NC, num_subcores=NS
    ),
    scratch_types=[
        pltpu.VMEM((ROWS_PER_W, CHUNK), jnp.int32),
        pltpu.VMEM((ROWS_PER_W, CHUNK), jnp.int32),
        pltpu.VMEM((2, CHUNK, H), jnp.float32),
        pltpu.VMEM((ZROWS, H), jnp.float32),
        pltpu.VMEM_SHARED((N_ACC, H), jnp.float32),
        pltpu.SemaphoreType.DMA,
        pltpu.SemaphoreType.DMA,
    ],
    compiler_params=pltpu.CompilerParams(use_tc_tiling_on_sc=False),
)

BLK = 1000
GRID = N // BLK


def _proj_body(x_ref, w_ref, o_ref):
    o_ref[:] = jnp.dot(x_ref[:], w_ref[:], preferred_element_type=jnp.float32)


_proj = pl.pallas_call(
    _proj_body,
    grid=(GRID,),
    in_specs=[
        pl.BlockSpec((BLK, F_IN), lambda i: (i, 0)),
        pl.BlockSpec((F_IN, H), lambda i: (0, 0)),
    ],
    out_specs=pl.BlockSpec((BLK, H), lambda i: (i, 0)),
    out_shape=jax.ShapeDtypeStruct((N, H), jnp.float32),
)


def _mlp1_body(y_ref, a0_ref, a1_ref, ba_ref, w_ref, bb_ref, g_ref, be_ref, o_ref):
    t = jnp.maximum(y_ref[:] + a0_ref[:] + a1_ref[:] + ba_ref[:], 0.0)
    h = jnp.dot(t, w_ref[:], preferred_element_type=jnp.float32) + bb_ref[:]
    scale = g_ref[:] * (1.0 / np.sqrt(1.0 + BN_EPS))
    o_ref[:] = jnp.maximum(h * scale + be_ref[:], 0.0)


_row = lambda i: (i, 0)
_fix = lambda i: (0, 0)

_mlp1 = pl.pallas_call(
    _mlp1_body,
    grid=(GRID,),
    in_specs=[
        pl.BlockSpec((BLK, H), _row),
        pl.BlockSpec((BLK, H), _row),
        pl.BlockSpec((BLK, H), _row),
        pl.BlockSpec((1, H), _fix),
        pl.BlockSpec((H, H), _fix),
        pl.BlockSpec((1, H), _fix),
        pl.BlockSpec((1, H), _fix),
        pl.BlockSpec((1, H), _fix),
    ],
    out_specs=pl.BlockSpec((BLK, H), _row),
    out_shape=jax.ShapeDtypeStruct((N, H), jnp.float32),
)


def _mlp2_body(h_ref, a0_ref, a1_ref, bid_ref, wa_ref, ba_ref, wb_ref, bb_ref,
               g_ref, be_ref, o_ref, acc_s, acc_c):
    j = pl.program_id(0)

    @pl.when(j == 0)
    def _():
        acc_s[:] = jnp.zeros_like(acc_s)
        acc_c[:] = jnp.zeros_like(acc_c)

    u = h_ref[:] + a0_ref[:] + a1_ref[:]
    t = jnp.maximum(jnp.dot(u, wa_ref[:], preferred_element_type=jnp.float32) + ba_ref[:], 0.0)
    h2 = jnp.dot(t, wb_ref[:], preferred_element_type=jnp.float32) + bb_ref[:]
    scale = g_ref[:] * (1.0 / np.sqrt(1.0 + BN_EPS))
    h2 = jnp.maximum(h2 * scale + be_ref[:], 0.0)

    oh = (bid_ref[:] == lax.broadcasted_iota(jnp.int32, (1, G), 1)).astype(jnp.float32)
    dims = (((0,), (0,)), ((), ()))
    acc_s[:] += lax.dot_general(oh, h2, dims, preferred_element_type=jnp.float32)
    acc_c[:] += lax.dot_general(oh, jnp.ones_like(h2), dims, preferred_element_type=jnp.float32)

    @pl.when(j == pl.num_programs(0) - 1)
    def _():
        o_ref[:] = acc_s[:] / jnp.maximum(acc_c[:], 1.0)


_mlp2_pool = pl.pallas_call(
    _mlp2_body,
    grid=(GRID,),
    in_specs=[
        pl.BlockSpec((BLK, H), _row),
        pl.BlockSpec((BLK, H), _row),
        pl.BlockSpec((BLK, H), _row),
        pl.BlockSpec((BLK, 1), _row),
        pl.BlockSpec((H, H), _fix),
        pl.BlockSpec((1, H), _fix),
        pl.BlockSpec((H, H), _fix),
        pl.BlockSpec((1, H), _fix),
        pl.BlockSpec((1, H), _fix),
        pl.BlockSpec((1, H), _fix),
    ],
    out_specs=pl.BlockSpec((G, H), _fix),
    out_shape=jax.ShapeDtypeStruct((G, H), jnp.float32),
    scratch_shapes=[
        pltpu.VMEM((G, H), jnp.float32),
        pltpu.VMEM((G, H), jnp.float32),
    ],
)


def kernel(x, ei, batch, W1a, b1a, W1b, b1b, g1, be1, W2a, b2a, W2b, b2b, g2, be2):
    src, dst = ei[0], ei[1]
    pad = E_PAD - E
    src_p = jnp.concatenate([src, jnp.zeros((pad,), jnp.int32)]).reshape(ROWS, CHUNK)
    dst_p = jnp.concatenate([dst, jnp.full((pad,), N, jnp.int32)]).reshape(ROWS, CHUNK)

    y = _proj(x, W1a)
    a1 = _seg_sum(y, src_p, dst_p)
    h1 = _mlp1(y, a1[0, :N], a1[1, :N], b1a.reshape(1, H), W1b, b1b.reshape(1, H),
               g1.reshape(1, H), be1.reshape(1, H))
    a2 = _seg_sum(h1, src_p, dst_p)
    out = _mlp2_pool(h1, a2[0, :N], a2[1, :N], batch.reshape(N, 1), W2a,
                     b2a.reshape(1, H), W2b, b2b.reshape(1, H),
                     g2.reshape(1, H), be2.reshape(1, H))
    return out


# speedup vs baseline: 6.2407x; 1.0692x over previous
import functools

import jax
import jax.numpy as jnp
import numpy as np
from jax import lax
from jax.experimental import pallas as pl
from jax.experimental.pallas import tpu as pltpu
from jax.experimental.pallas import tpu_sc as plsc

N, E, F_IN, H, G = 10000, 320000, 128, 64, 64
BN_EPS = 1e-5

NC, NS = 2, 16
NW = NC * NS
CHUNK = 128
ROWS_PER_W = 80
ROWS = NW * ROWS_PER_W
RPW0, RPW1 = 104, 56
RPW_MAX = max(RPW0, RPW1)
ROWS_ALLOC = NS * (RPW0 + RPW1)
E_PAD = ROWS_ALLOC * CHUNK
N_ACC = 10112
ZROWS = N_ACC // NS


def _seg_body(y_hbm, src_hbm, dst_hbm, out_hbm, src_v, dst_v, rows_v, zbuf, acc, sem0, sem1):
    c = lax.axis_index("c")
    s = lax.axis_index("s")
    wid = c * NS + s

    def _zrow(r, carry):
        for k in range(H // 16):
            zbuf[r, pl.ds(16 * k, 16)] = jnp.zeros((16,), jnp.float32)
        return carry

    lax.fori_loop(0, ZROWS, _zrow, 0)
    pltpu.sync_copy(zbuf, acc.at[pl.ds(s * ZROWS, ZROWS)])

    plsc.subcore_barrier()

    sems = (sem0, sem1)
    bufs = (rows_v.at[0], rows_v.at[1])

    def _gather(j, b):
        pltpu.make_async_copy(y_hbm.at[src_v.at[j]], bufs[b], sems[b]).start()

    def _gwait(b):
        pltpu.make_async_copy(y_hbm.at[src_v.at[0]], bufs[b], sems[b]).wait()

    def _run(rpw, base):
        pltpu.sync_copy(src_hbm.at[pl.ds(base, rpw)], src_v.at[pl.ds(0, rpw)])
        pltpu.sync_copy(dst_hbm.at[pl.ds(base, rpw)], dst_v.at[pl.ds(0, rpw)])
        _gather(0, 0)

        def _step(j2, carry):
            for b in range(2):
                j = 2 * j2 + b
                _gwait(b)
                nxt = j + 1

                @pl.when(nxt < rpw)
                def _():
                    _gather(nxt, 1 - b)

                pltpu.sync_copy(bufs[b], acc.at[dst_v.at[j]], add=True)
            return carry

        lax.fori_loop(0, rpw // 2, _step, 0)

    @pl.when(c == 0)
    def _():
        _run(RPW0, s * RPW0)

    @pl.when(c == 1)
    def _():
        _run(RPW1, NS * RPW0 + s * RPW1)

    plsc.subcore_barrier()

    pltpu.sync_copy(acc.at[pl.ds(s * ZROWS, ZROWS)], out_hbm.at[c].at[pl.ds(s * ZROWS, ZROWS)])


_seg_sum = pl.kernel(
    _seg_body,
    out_type=jax.ShapeDtypeStruct((NC, N_ACC, H), jnp.float32),
    mesh=plsc.VectorSubcoreMesh(
        core_axis_name="c", subcore_axis_name="s", num_cores=NC, num_subcores=NS
    ),
    scratch_types=[
        pltpu.VMEM((RPW_MAX, CHUNK), jnp.int32),
        pltpu.VMEM((RPW_MAX, CHUNK), jnp.int32),
        pltpu.VMEM((2, CHUNK, H), jnp.float32),
        pltpu.VMEM((ZROWS, H), jnp.float32),
        pltpu.VMEM_SHARED((N_ACC, H), jnp.float32),
        pltpu.SemaphoreType.DMA,
        pltpu.SemaphoreType.DMA,
    ],
    compiler_params=pltpu.CompilerParams(use_tc_tiling_on_sc=False),
)

BLK = 1000
GRID = N // BLK


def _proj_body(x_ref, w_ref, o_ref):
    o_ref[:] = jnp.dot(x_ref[:], w_ref[:], preferred_element_type=jnp.float32)


_proj = pl.pallas_call(
    _proj_body,
    grid=(GRID,),
    in_specs=[
        pl.BlockSpec((BLK, F_IN), lambda i: (i, 0)),
        pl.BlockSpec((F_IN, H), lambda i: (0, 0)),
    ],
    out_specs=pl.BlockSpec((BLK, H), lambda i: (i, 0)),
    out_shape=jax.ShapeDtypeStruct((N, H), jnp.float32),
)


def _mlp1_body(y_ref, a0_ref, a1_ref, ba_ref, w_ref, bb_ref, g_ref, be_ref, o_ref):
    t = jnp.maximum(y_ref[:] + a0_ref[:] + a1_ref[:] + ba_ref[:], 0.0)
    h = jnp.dot(t, w_ref[:], preferred_element_type=jnp.float32) + bb_ref[:]
    scale = g_ref[:] * (1.0 / np.sqrt(1.0 + BN_EPS))
    o_ref[:] = jnp.maximum(h * scale + be_ref[:], 0.0)


_row = lambda i: (i, 0)
_fix = lambda i: (0, 0)

_mlp1 = pl.pallas_call(
    _mlp1_body,
    grid=(GRID,),
    in_specs=[
        pl.BlockSpec((BLK, H), _row),
        pl.BlockSpec((BLK, H), _row),
        pl.BlockSpec((BLK, H), _row),
        pl.BlockSpec((1, H), _fix),
        pl.BlockSpec((H, H), _fix),
        pl.BlockSpec((1, H), _fix),
        pl.BlockSpec((1, H), _fix),
        pl.BlockSpec((1, H), _fix),
    ],
    out_specs=pl.BlockSpec((BLK, H), _row),
    out_shape=jax.ShapeDtypeStruct((N, H), jnp.float32),
)


def _mlp2_body(h_ref, a0_ref, a1_ref, bid_ref, wa_ref, ba_ref, wb_ref, bb_ref,
               g_ref, be_ref, o_ref, acc_s, acc_c):
    j = pl.program_id(0)

    @pl.when(j == 0)
    def _():
        acc_s[:] = jnp.zeros_like(acc_s)
        acc_c[:] = jnp.zeros_like(acc_c)

    u = h_ref[:] + a0_ref[:] + a1_ref[:]
    t = jnp.maximum(jnp.dot(u, wa_ref[:], preferred_element_type=jnp.float32) + ba_ref[:], 0.0)
    h2 = jnp.dot(t, wb_ref[:], preferred_element_type=jnp.float32) + bb_ref[:]
    scale = g_ref[:] * (1.0 / np.sqrt(1.0 + BN_EPS))
    h2 = jnp.maximum(h2 * scale + be_ref[:], 0.0)

    oh = (bid_ref[:] == lax.broadcasted_iota(jnp.int32, (1, G), 1)).astype(jnp.float32)
    dims = (((0,), (0,)), ((), ()))
    acc_s[:] += lax.dot_general(oh, h2, dims, preferred_element_type=jnp.float32)
    acc_c[:] += lax.dot_general(oh, jnp.ones_like(h2), dims, preferred_element_type=jnp.float32)

    @pl.when(j == pl.num_programs(0) - 1)
    def _():
        o_ref[:] = acc_s[:] / jnp.maximum(acc_c[:], 1.0)


_mlp2_pool = pl.pallas_call(
    _mlp2_body,
    grid=(GRID,),
    in_specs=[
        pl.BlockSpec((BLK, H), _row),
        pl.BlockSpec((BLK, H), _row),
        pl.BlockSpec((BLK, H), _row),
        pl.BlockSpec((BLK, 1), _row),
        pl.BlockSpec((H, H), _fix),
        pl.BlockSpec((1, H), _fix),
        pl.BlockSpec((H, H), _fix),
        pl.BlockSpec((1, H), _fix),
        pl.BlockSpec((1, H), _fix),
        pl.BlockSpec((1, H), _fix),
    ],
    out_specs=pl.BlockSpec((G, H), _fix),
    out_shape=jax.ShapeDtypeStruct((G, H), jnp.float32),
    scratch_shapes=[
        pltpu.VMEM((G, H), jnp.float32),
        pltpu.VMEM((G, H), jnp.float32),
    ],
)


def kernel(x, ei, batch, W1a, b1a, W1b, b1b, g1, be1, W2a, b2a, W2b, b2b, g2, be2):
    src, dst = ei[0], ei[1]
    pad = E_PAD - E
    src_p = jnp.concatenate([src, jnp.zeros((pad,), jnp.int32)]).reshape(ROWS_ALLOC, CHUNK)
    dst_p = jnp.concatenate([dst, jnp.full((pad,), N, jnp.int32)]).reshape(ROWS_ALLOC, CHUNK)

    y = _proj(x, W1a)
    a1 = _seg_sum(y, src_p, dst_p)
    h1 = _mlp1(y, a1[0, :N], a1[1, :N], b1a.reshape(1, H), W1b, b1b.reshape(1, H),
               g1.reshape(1, H), be1.reshape(1, H))
    a2 = _seg_sum(h1, src_p, dst_p)
    out = _mlp2_pool(h1, a2[0, :N], a2[1, :N], batch.reshape(N, 1), W2a,
                     b2a.reshape(1, H), W2b, b2b.reshape(1, H),
                     g2.reshape(1, H), be2.reshape(1, H))
    return out


# speedup vs baseline: 6.6649x; 1.0680x over previous
import functools

import jax
import jax.numpy as jnp
import numpy as np
from jax import lax
from jax.experimental import pallas as pl
from jax.experimental.pallas import tpu as pltpu
from jax.experimental.pallas import tpu_sc as plsc

N, E, F_IN, H, G = 10000, 320000, 128, 64, 64
BN_EPS = 1e-5

NC, NS = 2, 16
NW = NC * NS
CHUNK = 128
ROWS_PER_W = 80
ROWS = NW * ROWS_PER_W
RPW0, RPW1 = 128, 32
RPW_MAX = max(RPW0, RPW1)
ROWS_ALLOC = NS * (RPW0 + RPW1)
E_PAD = ROWS_ALLOC * CHUNK
N_ACC = 10112
ZROWS = N_ACC // NS


def _seg_body(y_hbm, src_hbm, dst_hbm, out_hbm, src_v, dst_v, rows_v, zbuf, acc, sem0, sem1):
    c = lax.axis_index("c")
    s = lax.axis_index("s")
    wid = c * NS + s

    def _zrow(r, carry):
        for k in range(H // 16):
            zbuf[r, pl.ds(16 * k, 16)] = jnp.zeros((16,), jnp.float32)
        return carry

    lax.fori_loop(0, ZROWS, _zrow, 0)
    pltpu.sync_copy(zbuf, acc.at[pl.ds(s * ZROWS, ZROWS)])

    plsc.subcore_barrier()

    sems = (sem0, sem1)
    bufs = (rows_v.at[0], rows_v.at[1])

    def _gather(j, b):
        pltpu.make_async_copy(y_hbm.at[src_v.at[j]], bufs[b], sems[b]).start()

    def _gwait(b):
        pltpu.make_async_copy(y_hbm.at[src_v.at[0]], bufs[b], sems[b]).wait()

    def _run(rpw, base):
        pltpu.sync_copy(src_hbm.at[pl.ds(base, rpw)], src_v.at[pl.ds(0, rpw)])
        pltpu.sync_copy(dst_hbm.at[pl.ds(base, rpw)], dst_v.at[pl.ds(0, rpw)])
        _gather(0, 0)

        def _step(j2, carry):
            for b in range(2):
                j = 2 * j2 + b
                _gwait(b)
                nxt = j + 1

                @pl.when(nxt < rpw)
                def _():
                    _gather(nxt, 1 - b)

                pltpu.sync_copy(bufs[b], acc.at[dst_v.at[j]], add=True)
            return carry

        lax.fori_loop(0, rpw // 2, _step, 0)

    @pl.when(c == 0)
    def _():
        _run(RPW0, s * RPW0)

    @pl.when(c == 1)
    def _():
        _run(RPW1, NS * RPW0 + s * RPW1)

    plsc.subcore_barrier()

    pltpu.sync_copy(acc.at[pl.ds(s * ZROWS, ZROWS)], out_hbm.at[c].at[pl.ds(s * ZROWS, ZROWS)])


_seg_sum = pl.kernel(
    _seg_body,
    out_type=jax.ShapeDtypeStruct((NC, N_ACC, H), jnp.float32),
    mesh=plsc.VectorSubcoreMesh(
        core_axis_name="c", subcore_axis_name="s", num_cores=NC, num_subcores=NS
    ),
    scratch_types=[
        pltpu.VMEM((RPW_MAX, CHUNK), jnp.int32),
        pltpu.VMEM((RPW_MAX, CHUNK), jnp.int32),
        pltpu.VMEM((2, CHUNK, H), jnp.float32),
        pltpu.VMEM((ZROWS, H), jnp.float32),
        pltpu.VMEM_SHARED((N_ACC, H), jnp.float32),
        pltpu.SemaphoreType.DMA,
        pltpu.SemaphoreType.DMA,
    ],
    compiler_params=pltpu.CompilerParams(use_tc_tiling_on_sc=False),
)

BLK = 1000
GRID = N // BLK


def _proj_body(x_ref, w_ref, o_ref):
    o_ref[:] = jnp.dot(x_ref[:], w_ref[:], preferred_element_type=jnp.float32)


_proj = pl.pallas_call(
    _proj_body,
    grid=(GRID,),
    in_specs=[
        pl.BlockSpec((BLK, F_IN), lambda i: (i, 0)),
        pl.BlockSpec((F_IN, H), lambda i: (0, 0)),
    ],
    out_specs=pl.BlockSpec((BLK, H), lambda i: (i, 0)),
    out_shape=jax.ShapeDtypeStruct((N, H), jnp.float32),
)


def _mlp1_body(y_ref, a0_ref, a1_ref, ba_ref, w_ref, bb_ref, g_ref, be_ref, o_ref):
    t = jnp.maximum(y_ref[:] + a0_ref[:] + a1_ref[:] + ba_ref[:], 0.0)
    h = jnp.dot(t, w_ref[:], preferred_element_type=jnp.float32) + bb_ref[:]
    scale = g_ref[:] * (1.0 / np.sqrt(1.0 + BN_EPS))
    o_ref[:] = jnp.maximum(h * scale + be_ref[:], 0.0)


_row = lambda i: (i, 0)
_fix = lambda i: (0, 0)

_mlp1 = pl.pallas_call(
    _mlp1_body,
    grid=(GRID,),
    in_specs=[
        pl.BlockSpec((BLK, H), _row),
        pl.BlockSpec((BLK, H), _row),
        pl.BlockSpec((BLK, H), _row),
        pl.BlockSpec((1, H), _fix),
        pl.BlockSpec((H, H), _fix),
        pl.BlockSpec((1, H), _fix),
        pl.BlockSpec((1, H), _fix),
        pl.BlockSpec((1, H), _fix),
    ],
    out_specs=pl.BlockSpec((BLK, H), _row),
    out_shape=jax.ShapeDtypeStruct((N, H), jnp.float32),
)


def _mlp2_body(h_ref, a0_ref, a1_ref, bid_ref, wa_ref, ba_ref, wb_ref, bb_ref,
               g_ref, be_ref, o_ref, acc_s, acc_c):
    j = pl.program_id(0)

    @pl.when(j == 0)
    def _():
        acc_s[:] = jnp.zeros_like(acc_s)
        acc_c[:] = jnp.zeros_like(acc_c)

    u = h_ref[:] + a0_ref[:] + a1_ref[:]
    t = jnp.maximum(jnp.dot(u, wa_ref[:], preferred_element_type=jnp.float32) + ba_ref[:], 0.0)
    h2 = jnp.dot(t, wb_ref[:], preferred_element_type=jnp.float32) + bb_ref[:]
    scale = g_ref[:] * (1.0 / np.sqrt(1.0 + BN_EPS))
    h2 = jnp.maximum(h2 * scale + be_ref[:], 0.0)

    oh = (bid_ref[:] == lax.broadcasted_iota(jnp.int32, (1, G), 1)).astype(jnp.float32)
    dims = (((0,), (0,)), ((), ()))
    acc_s[:] += lax.dot_general(oh, h2, dims, preferred_element_type=jnp.float32)
    acc_c[:] += lax.dot_general(oh, jnp.ones_like(h2), dims, preferred_element_type=jnp.float32)

    @pl.when(j == pl.num_programs(0) - 1)
    def _():
        o_ref[:] = acc_s[:] / jnp.maximum(acc_c[:], 1.0)


_mlp2_pool = pl.pallas_call(
    _mlp2_body,
    grid=(GRID,),
    in_specs=[
        pl.BlockSpec((BLK, H), _row),
        pl.BlockSpec((BLK, H), _row),
        pl.BlockSpec((BLK, H), _row),
        pl.BlockSpec((BLK, 1), _row),
        pl.BlockSpec((H, H), _fix),
        pl.BlockSpec((1, H), _fix),
        pl.BlockSpec((H, H), _fix),
        pl.BlockSpec((1, H), _fix),
        pl.BlockSpec((1, H), _fix),
        pl.BlockSpec((1, H), _fix),
    ],
    out_specs=pl.BlockSpec((G, H), _fix),
    out_shape=jax.ShapeDtypeStruct((G, H), jnp.float32),
    scratch_shapes=[
        pltpu.VMEM((G, H), jnp.float32),
        pltpu.VMEM((G, H), jnp.float32),
    ],
)


def kernel(x, ei, batch, W1a, b1a, W1b, b1b, g1, be1, W2a, b2a, W2b, b2b, g2, be2):
    src, dst = ei[0], ei[1]
    pad = E_PAD - E
    src_p = jnp.concatenate([src, jnp.zeros((pad,), jnp.int32)]).reshape(ROWS_ALLOC, CHUNK)
    dst_p = jnp.concatenate([dst, jnp.full((pad,), N, jnp.int32)]).reshape(ROWS_ALLOC, CHUNK)

    y = _proj(x, W1a)
    a1 = _seg_sum(y, src_p, dst_p)
    h1 = _mlp1(y, a1[0, :N], a1[1, :N], b1a.reshape(1, H), W1b, b1b.reshape(1, H),
               g1.reshape(1, H), be1.reshape(1, H))
    a2 = _seg_sum(h1, src_p, dst_p)
    out = _mlp2_pool(h1, a2[0, :N], a2[1, :N], batch.reshape(N, 1), W2a,
                     b2a.reshape(1, H), W2b, b2b.reshape(1, H),
                     g2.reshape(1, H), be2.reshape(1, H))
    return out
